# Initial kernel scaffold; baseline (speedup 1.0000x reference)
import jax
import jax.numpy as jnp
from jax.experimental import pallas as pl


def kernel(x):
    raise NotImplementedError("write your pallas kernel here")


# speedup vs baseline: 10.8922x; 10.8922x over previous
import functools

import jax
import jax.numpy as jnp
from jax import lax
from jax.experimental import pallas as pl
from jax.experimental.pallas import tpu as pltpu
from jax.experimental.pallas import tpu_sc as plsc

_NC = 2
_NS = 16
_NW = _NC * _NS
_CH = 32768


def _groupsort_flat(xflat):
    n = xflat.shape[0]
    per_w = n // _NW
    n_ch = per_w // _CH
    assert per_w % _CH == 0
    mesh = plsc.VectorSubcoreMesh(core_axis_name="c", subcore_axis_name="s")

    @functools.partial(
        pl.kernel,
        mesh=mesh,
        out_type=jax.ShapeDtypeStruct((n,), jnp.float32),
        scratch_types=[
            pltpu.VMEM((_CH,), jnp.float32),
        ],
        compiler_params=pltpu.CompilerParams(needs_layout_passes=False),
    )
    def k(x_hbm, out_hbm, buf):
        wid = lax.axis_index("s") * _NC + lax.axis_index("c")
        base_w = wid * per_w
        lane = lax.iota(jnp.int32, 16)
        i_a = lane * 4

        def chunk_body(g, carry):
            off = base_w + g * _CH
            pltpu.sync_copy(x_hbm.at[pl.ds(off, _CH)], buf)

            def span_body(j, c2):
                i0 = i_a + j * 64
                a = plsc.load_gather(buf, [i0])
                b = plsc.load_gather(buf, [i0 + 1])
                c = plsc.load_gather(buf, [i0 + 2])
                d = plsc.load_gather(buf, [i0 + 3])
                hi1 = jnp.maximum(a, b)
                lo1 = jnp.minimum(a, b)
                hi2 = jnp.maximum(c, d)
                lo2 = jnp.minimum(c, d)
                s0 = jnp.maximum(hi1, hi2)
                m1 = jnp.minimum(hi1, hi2)
                s3 = jnp.minimum(lo1, lo2)
                m2 = jnp.maximum(lo1, lo2)
                s1 = jnp.maximum(m1, m2)
                s2 = jnp.minimum(m1, m2)
                plsc.store_scatter(buf, [i0], s0)
                plsc.store_scatter(buf, [i0 + 1], s1)
                plsc.store_scatter(buf, [i0 + 2], s2)
                plsc.store_scatter(buf, [i0 + 3], s3)
                return c2

            lax.fori_loop(0, _CH // 64, span_body, 0)
            pltpu.sync_copy(buf, out_hbm.at[pl.ds(off, _CH)])
            return carry

        lax.fori_loop(0, n_ch, chunk_body, 0)

    return k(xflat)


def kernel(x):
    b, c = x.shape
    assert c == 4096 and b == 8192
    return _groupsort_flat(x.reshape(-1)).reshape(b, c)


# speedup vs baseline: 15.6719x; 1.4388x over previous
import functools

import jax
import jax.numpy as jnp
from jax import lax
from jax.experimental import pallas as pl
from jax.experimental.pallas import tpu as pltpu
from jax.experimental.pallas import tpu_sc as plsc

_NC = 2
_NS = 16
_NW = _NC * _NS
_CH = 32768
_NBUF = 3


def _groupsort_flat(xflat):
    n = xflat.shape[0]
    per_w = n // _NW
    n_ch = per_w // _CH
    assert per_w % _CH == 0 and n_ch >= _NBUF
    n_spans = _CH // 64
    mesh = plsc.VectorSubcoreMesh(core_axis_name="c", subcore_axis_name="s")

    @functools.partial(
        pl.kernel,
        mesh=mesh,
        out_type=jax.ShapeDtypeStruct((n,), jnp.float32),
        scratch_types=[
            *([pltpu.VMEM((_CH,), jnp.float32)] * _NBUF),
            *([pltpu.SemaphoreType.DMA] * (2 * _NBUF)),
        ],
        compiler_params=pltpu.CompilerParams(needs_layout_passes=False),
    )
    def k(x_hbm, out_hbm, b0, b1, b2, si0, si1, si2, so0, so1, so2):
        bufs = (b0, b1, b2)
        isems = (si0, si1, si2)
        osems = (so0, so1, so2)
        wid = lax.axis_index("s") * _NC + lax.axis_index("c")
        base_w = wid * per_w
        lane = lax.iota(jnp.int32, 16)
        i_a = lane * 4

        def sort_chunk(buf):
            @plsc.parallel_loop(0, n_spans, unroll=4)
            def span_body(j):
                i0 = i_a + j * 64
                a = plsc.load_gather(buf, [i0])
                b = plsc.load_gather(buf, [i0 + 1])
                c = plsc.load_gather(buf, [i0 + 2])
                d = plsc.load_gather(buf, [i0 + 3])
                hi1 = jnp.maximum(a, b)
                lo1 = jnp.minimum(a, b)
                hi2 = jnp.maximum(c, d)
                lo2 = jnp.minimum(c, d)
                s0 = jnp.maximum(hi1, hi2)
                m1 = jnp.minimum(hi1, hi2)
                s3 = jnp.minimum(lo1, lo2)
                m2 = jnp.maximum(lo1, lo2)
                s1 = jnp.maximum(m1, m2)
                s2 = jnp.minimum(m1, m2)
                plsc.store_scatter(buf, [i0], s0)
                plsc.store_scatter(buf, [i0 + 1], s1)
                plsc.store_scatter(buf, [i0 + 2], s2)
                plsc.store_scatter(buf, [i0 + 3], s3)

        def in_slice(g):
            return x_hbm.at[pl.ds(base_w + g * _CH, _CH)]

        def out_slice(g):
            return out_hbm.at[pl.ds(base_w + g * _CH, _CH)]

        in_h = {}
        out_h = {}
        in_h[0] = pltpu.async_copy(in_slice(0), bufs[0], isems[0])
        in_h[1] = pltpu.async_copy(in_slice(1), bufs[1], isems[1])
        for g in range(n_ch):
            b = g % _NBUF
            in_h[g].wait()
            sort_chunk(bufs[b])
            out_h[g] = pltpu.async_copy(bufs[b], out_slice(g), osems[b])
            nx = g + 2
            if nx < n_ch:
                if nx >= _NBUF:
                    out_h[nx - _NBUF].wait()
                in_h[nx] = pltpu.async_copy(
                    in_slice(nx), bufs[nx % _NBUF], isems[nx % _NBUF])
        for g in range(max(0, n_ch - _NBUF), n_ch):
            out_h[g].wait()

    return k(xflat)


def kernel(x):
    b, c = x.shape
    assert c == 4096 and b == 8192
    return _groupsort_flat(x.reshape(-1)).reshape(b, c)
